# baseline (speedup 1.0000x reference)
import functools

import jax
import jax.numpy as jnp
from jax import lax
from jax.experimental import pallas as pl
from jax.experimental.pallas import tpu as pltpu
from jax.experimental.pallas import tpu_sc as plsc

MARGIN_ = 1.0
NC, NS, L = 2, 16, 16
NW = NC * NS
PAIRS = 16384
PW = PAIRS // NW
STEP = 128
NSTEPS = PW // STEP
JROWS = 8
D = 64
DP = 128


def _lane_perm(x, idx):
    dnums = lax.GatherDimensionNumbers(
        offset_dims=(), collapsed_slice_dims=(0,), start_index_map=(0,))
    return lax.gather(x, idx[:, None], dnums, (1,),
                      mode=lax.GatherScatterMode.PROMISE_IN_BOUNDS)


def _sc_loss_kernel(ent_hbm, rel_hbm, ps_h, pr_h, pt_h, ns_h, nr_h, nt_h,
                    out_hbm,
                    ps_v, pr_v, pt_v, ns_v, nr_v, nt_v,
                    fs_v, fr_v, ft_v, gs_v, gr_v, gt_v,
                    r_ps, r_pr, r_pt, r_ns, r_nr, r_nt,
                    acc_v, sem):
    wid = lax.axis_index("s") * NC + lax.axis_index("c")

    pltpu.sync_copy(ps_h.at[wid], ps_v)
    pltpu.sync_copy(pr_h.at[wid], pr_v)
    pltpu.sync_copy(pt_h.at[wid], pt_v)
    pltpu.sync_copy(ns_h.at[wid], ns_v)
    pltpu.sync_copy(nr_h.at[wid], nr_v)
    pltpu.sync_copy(nt_h.at[wid], nt_v)

    for ov, pv in ((ps_v, fs_v), (pr_v, fr_v), (pt_v, ft_v),
                   (ns_v, gs_v), (nr_v, gr_v), (nt_v, gt_v)):
        def shift_row(j, _):
            for c in range(STEP // L):
                sl = pl.ds(c * L, L)
                pv[j, sl] = lax.shift_right_logical(ov[j, sl], 1)
            return 0
        lax.fori_loop(0, NSTEPS, shift_row, 0)

    lanes = lax.iota(jnp.int32, L)
    acc = jnp.zeros((L,), jnp.float32)
    for j in range(NSTEPS):
        cps = pltpu.async_copy(ent_hbm.at[fs_v.at[j]], r_ps, sem)
        cpr = pltpu.async_copy(rel_hbm.at[fr_v.at[j]], r_pr, sem)
        cpt = pltpu.async_copy(ent_hbm.at[ft_v.at[j]], r_pt, sem)
        cns = pltpu.async_copy(ent_hbm.at[gs_v.at[j]], r_ns, sem)
        cnr = pltpu.async_copy(rel_hbm.at[gr_v.at[j]], r_nr, sem)
        cnt = pltpu.async_copy(ent_hbm.at[gt_v.at[j]], r_nt, sem)
        for c in (cps, cpr, cpt, cns, cnr, cnt):
            c.wait()

        def pair_block(pb, acc, j=j):
            chunks = [ov[j, pl.ds(pb * L, L)]
                      for ov in (ps_v, pr_v, pt_v, ns_v, nr_v, nt_v)]
            for u in range(L):
                p = pb * L + u
                offs = [lax.bitwise_and(ch[u], 1) * D for ch in chunks]
                d = jnp.zeros((L,), jnp.float32)
                for k in range(D // L):
                    b = k * L
                    xp = jnp.abs(r_ps[p, pl.ds(offs[0] + b, L)]
                                 + r_pr[p, pl.ds(offs[1] + b, L)]
                                 - r_pt[p, pl.ds(offs[2] + b, L)])
                    xn = jnp.abs(r_ns[p, pl.ds(offs[3] + b, L)]
                                 + r_nr[p, pl.ds(offs[4] + b, L)]
                                 - r_nt[p, pl.ds(offs[5] + b, L)])
                    d = d + (xp - xn)
                for sh in (8, 4, 2, 1):
                    d = d + _lane_perm(d, lanes ^ sh)
                acc = acc + jnp.maximum(d + MARGIN_, 0.0)
            return acc

        acc = lax.fori_loop(0, STEP // L, pair_block, acc)

    acc_v[pl.ds(0, L)] = jnp.where(lanes == 0, acc, 0.0)
    for z in range(1, DP // L):
        acc_v[pl.ds(z * L, L)] = jnp.zeros((L,), jnp.float32)
    pltpu.sync_copy(acc_v, out_hbm.at[wid])


@jax.jit
def kernel(train_indices, ent_embeds, rel_embeds):
    idx = train_indices.astype(jnp.int32)
    pos = idx[:PAIRS]
    neg = idx[PAIRS:]
    blocks = []
    for c in (pos[:, 0], pos[:, 1], pos[:, 2],
              neg[:, 0], neg[:, 1], neg[:, 2]):
        b = c.reshape(NW, NSTEPS, STEP)
        b = jnp.pad(b, ((0, 0), (0, JROWS - NSTEPS), (0, 0)))
        blocks.append(b)

    ent2 = ent_embeds.reshape(500000, DP)
    rel2 = rel_embeds.reshape(500000, DP)

    mesh = plsc.VectorSubcoreMesh(core_axis_name="c", subcore_axis_name="s")
    run = functools.partial(
        pl.kernel,
        mesh=mesh,
        out_type=jax.ShapeDtypeStruct((NW, DP), jnp.float32),
        scratch_types=(
            [pltpu.VMEM((JROWS, STEP), jnp.int32)] * 12
            + [pltpu.VMEM((STEP, DP), jnp.float32)] * 6
            + [pltpu.VMEM((DP,), jnp.float32), pltpu.SemaphoreType.DMA]
        ),
    )(_sc_loss_kernel)
    partials = run(ent2, rel2, *blocks)
    return jnp.sum(partials) / PAIRS
